# baseline (speedup 1.0000x reference)
import functools
import math

import jax
import jax.numpy as jnp
from jax import lax
from jax.experimental import pallas as pl
from jax.experimental.pallas import tpu as pltpu
from jax.experimental.pallas import tpu_sc as plsc

L = 8
T = 16384
FP = 2
NUM_VIEW = 16
NG = 8
HASH_K = -1640531535

NC, NS, LANES = 2, 16, 16
NW = NC * NS
C = 512
NP = 1015808
PW = NP // NW
ITERS = PW // C
IT0 = 82
IT1 = 2 * ITERS - IT0
GROUPS = C // LANES
ROWS = 4 * L * C
NSEG = 64
SEG = ROWS // NSEG

_BFAC = math.exp((math.log(512.0) - math.log(16.0)) / (L - 1))
RES = [float(math.floor(16.0 * (_BFAC ** l))) for l in range(L)]

CORNERS = ((0, 0), (0, 1), (1, 0), (1, 1))


def _encode_sc(x0p, x1p, vp, tab1d):
    mesh = plsc.VectorSubcoreMesh(core_axis_name="c", subcore_axis_name="s")

    @functools.partial(
        pl.kernel,
        mesh=mesh,
        compiler_params=pltpu.CompilerParams(needs_layout_passes=False),
        out_type=jax.ShapeDtypeStruct((2 * L, NP), jnp.float32),
        scratch_types=[
            pltpu.VMEM((2, C), jnp.float32),
            pltpu.VMEM((2, C), jnp.float32),
            pltpu.VMEM((2, C), jnp.int32),
            pltpu.VMEM((ROWS,), jnp.int32),
            pltpu.VMEM((ROWS,), jnp.int32),
            pltpu.VMEM((2, ROWS), jnp.float32),
            pltpu.VMEM((ROWS,), jnp.int32),
            pltpu.VMEM((ROWS,), jnp.int32),
            pltpu.VMEM((2, 2 * L, C), jnp.float32),
            pltpu.SemaphoreType.DMA,
            pltpu.SemaphoreType.DMA,
            pltpu.SemaphoreType.DMA,
            pltpu.SemaphoreType.DMA,
        ],
    )
    def enc(x0h, x1h, vh, tabh, outh, x0v, x1v, vv, idxv0, idxv1, wv,
            rowsv0, rowsv1, hv, gsem0, gsem1, hsem0, hsem1):
        cid = lax.axis_index("c")
        sid = lax.axis_index("s")
        my_iters = jnp.where(cid == 0, IT0, IT1)
        base0 = jnp.where(cid == 0, sid * (IT0 * C),
                          NS * (IT0 * C) + sid * (IT1 * C))
        gsems = (gsem0, gsem1)
        hsems = (hsem0, hsem1)
        idxvs = (idxv0, idxv1)
        rowsvs = (rowsv0, rowsv1)

        def phase1(k, par):
            base = base0 + k * C
            pltpu.sync_copy(x0h.at[pl.ds(base, C)], x0v.at[par])
            pltpu.sync_copy(x1h.at[pl.ds(base, C)], x1v.at[par])
            pltpu.sync_copy(vh.at[pl.ds(base, C)], vv.at[par])

            def idx_body(g, c2):
                p0 = g * LANES
                xa = x0v[par, pl.ds(p0, LANES)]
                xb = x1v[par, pl.ds(p0, LANES)]
                vrow = vv[par, pl.ds(p0, LANES)] * (L * T)
                for l in range(L):
                    pa = xa * RES[l]
                    pb = xb * RES[l]
                    ia = pa.astype(jnp.int32)
                    ib = pb.astype(jnp.int32)
                    fa = pa - ia.astype(jnp.float32)
                    fb = pb - ib.astype(jnp.float32)
                    ga = 1.0 - fa
                    gb = 1.0 - fb
                    lb = vrow + l * T
                    for ci, (dx, dy) in enumerate(CORNERS):
                        cx = ia + dx if dx else ia
                        cy = ib + dy if dy else ib
                        h = (cx ^ (cy * HASH_K)) & (T - 1)
                        P = (l * 4 + ci) * C + p0
                        idxvs[par][pl.ds(P, LANES)] = lb + h
                        wx = fa if dx else ga
                        wy = fb if dy else gb
                        wv[par, pl.ds(P, LANES)] = wx * wy
                return c2

            lax.fori_loop(0, GROUPS, idx_body, 0)
            for j in range(NSEG):
                pltpu.async_copy(
                    tabh.at[idxvs[par].at[pl.ds(j * SEG, SEG)]],
                    rowsvs[par].at[pl.ds(j * SEG, SEG)],
                    gsems[par])

        def phase2(k, par):
            base = base0 + k * C
            for j in range(NSEG):
                pltpu.make_async_copy(
                    tabh.at[idxvs[par].at[pl.ds(j * SEG, SEG)]],
                    rowsvs[par].at[pl.ds(j * SEG, SEG)],
                    gsems[par]).wait()

            himask = jnp.full((LANES,), -65536, jnp.int32)

            def acc_body(g, c2):
                p0 = g * LANES
                for l in range(L):
                    a0 = jnp.zeros((LANES,), jnp.float32)
                    a1 = jnp.zeros((LANES,), jnp.float32)
                    for ci in range(4):
                        P = (l * 4 + ci) * C + p0
                        v = rowsvs[par][pl.ds(P, LANES)]
                        f0 = plsc.bitcast(v << 16, jnp.float32)
                        f1 = plsc.bitcast(v & himask, jnp.float32)
                        w = wv[par, pl.ds(P, LANES)]
                        a0 = a0 + f0 * w
                        a1 = a1 + f1 * w
                    hv[par, 2 * l, pl.ds(p0, LANES)] = a0
                    hv[par, 2 * l + 1, pl.ds(p0, LANES)] = a1
                return c2

            @pl.when(k >= 2)
            def _():
                pltpu.make_async_copy(
                    hv.at[par],
                    outh.at[:, pl.ds(base - 2 * C, C)],
                    hsems[par]).wait()

            lax.fori_loop(0, GROUPS, acc_body, 0)
            pltpu.async_copy(hv.at[par], outh.at[:, pl.ds(base, C)],
                             hsems[par])

        phase1(0, 0)

        def body(i2, carry):
            a = 2 * i2
            phase1(a + 1, 1)
            phase2(a, 0)

            @pl.when(a + 2 < my_iters)
            def _():
                phase1(a + 2, 0)

            phase2(a + 1, 1)
            return carry

        lax.fori_loop(0, my_iters // 2, body, 0)
        pltpu.make_async_copy(
            hv.at[0], outh.at[:, pl.ds(base0 + (my_iters - 2) * C, C)],
            hsems[0]).wait()
        pltpu.make_async_copy(
            hv.at[1], outh.at[:, pl.ds(base0 + (my_iters - 1) * C, C)],
            hsems[1]).wait()

    return enc(x0p, x1p, vp, tab1d)


def _mlp_tc(HT, W1, b1, W2, b2, n):
    BLK = 8192
    nblk = (n + BLK - 1) // BLK

    def mlp_body(h_ref, w1_ref, b1_ref, w2_ref, b2_ref,
                 mu_ref, inv_ref, wt_ref, ho_ref):
        hT = h_ref[...]
        h1 = lax.dot_general(w1_ref[...], hT, (((0,), (0,)), ((), ())),
                             preferred_element_type=jnp.float32)
        h1 = h1 + b1_ref[...]
        g = jnp.exp(h1 * h1 * (-50.0))
        raw = lax.dot_general(w2_ref[...], g, (((0,), (0,)), ((), ())),
                              preferred_element_type=jnp.float32)
        raw = raw + b2_ref[...]
        wt_ref[...] = jnp.exp(raw[0:NG, :])
        mu_ref[...] = jax.nn.sigmoid(raw[NG:2 * NG, :])
        inv_ref[...] = jnp.exp(raw[2 * NG:3 * NG, :])
        ho_ref[...] = hT

    return pl.pallas_call(
        mlp_body,
        grid=(nblk,),
        in_specs=[
            pl.BlockSpec((2 * L, BLK), lambda i: (0, i)),
            pl.BlockSpec((2 * L, 32), lambda i: (0, 0)),
            pl.BlockSpec((32, 1), lambda i: (0, 0)),
            pl.BlockSpec((32, 3 * NG), lambda i: (0, 0)),
            pl.BlockSpec((3 * NG, 1), lambda i: (0, 0)),
        ],
        out_specs=[
            pl.BlockSpec((NG, BLK), lambda i: (0, i)),
            pl.BlockSpec((NG, BLK), lambda i: (0, i)),
            pl.BlockSpec((NG, BLK), lambda i: (0, i)),
            pl.BlockSpec((2 * L, BLK), lambda i: (0, i)),
        ],
        out_shape=[
            jax.ShapeDtypeStruct((NG, n), jnp.float32),
            jax.ShapeDtypeStruct((NG, n), jnp.float32),
            jax.ShapeDtypeStruct((NG, n), jnp.float32),
            jax.ShapeDtypeStruct((2 * L, n), jnp.float32),
        ],
    )(HT, W1, b1.reshape(32, 1), W2, b2.reshape(3 * NG, 1))


def kernel(x, hashidxs, tables, W1, b1, W2, b2):
    n = x.shape[0]
    vidx = hashidxs.astype(jnp.int32)
    x0p = jnp.zeros((NP,), jnp.float32).at[:n].set(x[:, 0])
    x1p = jnp.zeros((NP,), jnp.float32).at[:n].set(x[:, 1])
    vp = jnp.zeros((NP,), jnp.int32).at[:n].set(vidx)
    tu = lax.bitcast_convert_type(tables.astype(jnp.bfloat16), jnp.uint16)
    tpk = tu.astype(jnp.uint32)
    tab1d = lax.bitcast_convert_type(
        tpk[..., 0] | (tpk[..., 1] << 16), jnp.int32).reshape(-1)
    HT = _encode_sc(x0p, x1p, vp, tab1d)
    muT, invT, wT, hT = _mlp_tc(HT, W1, b1, W2, b2, n)
    return (muT.T, invT.T, wT.T, hT.T)


# speedup vs baseline: 1.0557x; 1.0557x over previous
import functools
import math

import jax
import jax.numpy as jnp
from jax import lax
from jax.experimental import pallas as pl
from jax.experimental.pallas import tpu as pltpu
from jax.experimental.pallas import tpu_sc as plsc

L = 8
T = 16384
FP = 2
NUM_VIEW = 16
NG = 8
HASH_K = -1640531535

NC, NS, LANES = 2, 16, 16
NW = NC * NS
C = 512
NP = 1015808
PW = NP // NW
ITERS = PW // C
IT0 = 82
IT1 = 2 * ITERS - IT0
GROUPS = C // LANES
ROWS = 4 * L * C
NSEG = 32
SEG = ROWS // NSEG

_BFAC = math.exp((math.log(512.0) - math.log(16.0)) / (L - 1))
RES = [float(math.floor(16.0 * (_BFAC ** l))) for l in range(L)]

CORNERS = ((0, 0), (0, 1), (1, 0), (1, 1))


def _encode_sc(x0p, x1p, vp, tab1d):
    mesh = plsc.VectorSubcoreMesh(core_axis_name="c", subcore_axis_name="s")

    @functools.partial(
        pl.kernel,
        mesh=mesh,
        compiler_params=pltpu.CompilerParams(needs_layout_passes=False),
        out_type=jax.ShapeDtypeStruct((2 * L, NP), jnp.float32),
        scratch_types=[
            pltpu.VMEM((2, C), jnp.float32),
            pltpu.VMEM((2, C), jnp.float32),
            pltpu.VMEM((2, C), jnp.int32),
            pltpu.VMEM((ROWS,), jnp.int32),
            pltpu.VMEM((ROWS,), jnp.int32),
            pltpu.VMEM((2, ROWS), jnp.float32),
            pltpu.VMEM((ROWS,), jnp.int32),
            pltpu.VMEM((ROWS,), jnp.int32),
            pltpu.VMEM((2, 2 * L, C), jnp.float32),
            pltpu.SemaphoreType.DMA,
            pltpu.SemaphoreType.DMA,
            pltpu.SemaphoreType.DMA,
            pltpu.SemaphoreType.DMA,
        ],
    )
    def enc(x0h, x1h, vh, tabh, outh, x0v, x1v, vv, idxv0, idxv1, wv,
            rowsv0, rowsv1, hv, gsem0, gsem1, hsem0, hsem1):
        cid = lax.axis_index("c")
        sid = lax.axis_index("s")
        my_iters = jnp.where(cid == 0, IT0, IT1)
        base0 = jnp.where(cid == 0, sid * (IT0 * C),
                          NS * (IT0 * C) + sid * (IT1 * C))
        gsems = (gsem0, gsem1)
        hsems = (hsem0, hsem1)
        idxvs = (idxv0, idxv1)
        rowsvs = (rowsv0, rowsv1)

        def phase1(k, par):
            base = base0 + k * C
            pltpu.sync_copy(x0h.at[pl.ds(base, C)], x0v.at[par])
            pltpu.sync_copy(x1h.at[pl.ds(base, C)], x1v.at[par])
            pltpu.sync_copy(vh.at[pl.ds(base, C)], vv.at[par])

            @plsc.parallel_loop(0, GROUPS, unroll=2)
            def idx_body(g):
                p0 = g * LANES
                xa = x0v[par, pl.ds(p0, LANES)]
                xb = x1v[par, pl.ds(p0, LANES)]
                vrow = vv[par, pl.ds(p0, LANES)] * (L * T)
                for l in range(L):
                    pa = xa * RES[l]
                    pb = xb * RES[l]
                    ia = pa.astype(jnp.int32)
                    ib = pb.astype(jnp.int32)
                    fa = pa - ia.astype(jnp.float32)
                    fb = pb - ib.astype(jnp.float32)
                    ga = 1.0 - fa
                    gb = 1.0 - fb
                    lb = vrow + l * T
                    for ci, (dx, dy) in enumerate(CORNERS):
                        cx = ia + dx if dx else ia
                        cy = ib + dy if dy else ib
                        h = (cx ^ (cy * HASH_K)) & (T - 1)
                        P = (l * 4 + ci) * C + p0
                        idxvs[par][pl.ds(P, LANES)] = lb + h
                        wx = fa if dx else ga
                        wy = fb if dy else gb
                        wv[par, pl.ds(P, LANES)] = wx * wy

            for j in range(NSEG):
                pltpu.async_copy(
                    tabh.at[idxvs[par].at[pl.ds(j * SEG, SEG)]],
                    rowsvs[par].at[pl.ds(j * SEG, SEG)],
                    gsems[par])

        def phase2(k, par):
            base = base0 + k * C
            for j in range(NSEG):
                pltpu.make_async_copy(
                    tabh.at[idxvs[par].at[pl.ds(j * SEG, SEG)]],
                    rowsvs[par].at[pl.ds(j * SEG, SEG)],
                    gsems[par]).wait()

            himask = jnp.full((LANES,), -65536, jnp.int32)

            def run_acc():
                @plsc.parallel_loop(0, GROUPS, unroll=2)
                def acc_body(g):
                    p0 = g * LANES
                    for l in range(L):
                        a0 = jnp.zeros((LANES,), jnp.float32)
                        a1 = jnp.zeros((LANES,), jnp.float32)
                        for ci in range(4):
                            P = (l * 4 + ci) * C + p0
                            v = rowsvs[par][pl.ds(P, LANES)]
                            f0 = plsc.bitcast(v << 16, jnp.float32)
                            f1 = plsc.bitcast(v & himask, jnp.float32)
                            w = wv[par, pl.ds(P, LANES)]
                            a0 = a0 + f0 * w
                            a1 = a1 + f1 * w
                        hv[par, 2 * l, pl.ds(p0, LANES)] = a0
                        hv[par, 2 * l + 1, pl.ds(p0, LANES)] = a1

            @pl.when(k >= 2)
            def _():
                pltpu.make_async_copy(
                    hv.at[par],
                    outh.at[:, pl.ds(base - 2 * C, C)],
                    hsems[par]).wait()

            run_acc()
            pltpu.async_copy(hv.at[par], outh.at[:, pl.ds(base, C)],
                             hsems[par])

        phase1(0, 0)

        def body(i2, carry):
            a = 2 * i2
            phase1(a + 1, 1)
            phase2(a, 0)

            @pl.when(a + 2 < my_iters)
            def _():
                phase1(a + 2, 0)

            phase2(a + 1, 1)
            return carry

        lax.fori_loop(0, my_iters // 2, body, 0)
        pltpu.make_async_copy(
            hv.at[0], outh.at[:, pl.ds(base0 + (my_iters - 2) * C, C)],
            hsems[0]).wait()
        pltpu.make_async_copy(
            hv.at[1], outh.at[:, pl.ds(base0 + (my_iters - 1) * C, C)],
            hsems[1]).wait()

    return enc(x0p, x1p, vp, tab1d)


def _mlp_tc(HT, W1, b1, W2, b2, n):
    BLK = 8192
    nblk = (n + BLK - 1) // BLK

    def mlp_body(h_ref, w1_ref, b1_ref, w2_ref, b2_ref,
                 mu_ref, inv_ref, wt_ref, ho_ref):
        hT = h_ref[...]
        h1 = lax.dot_general(w1_ref[...], hT, (((0,), (0,)), ((), ())),
                             preferred_element_type=jnp.float32)
        h1 = h1 + b1_ref[...]
        g = jnp.exp(h1 * h1 * (-50.0))
        raw = lax.dot_general(w2_ref[...], g, (((0,), (0,)), ((), ())),
                              preferred_element_type=jnp.float32)
        raw = raw + b2_ref[...]
        wt_ref[...] = jnp.exp(raw[0:NG, :])
        mu_ref[...] = jax.nn.sigmoid(raw[NG:2 * NG, :])
        inv_ref[...] = jnp.exp(raw[2 * NG:3 * NG, :])
        ho_ref[...] = hT

    return pl.pallas_call(
        mlp_body,
        grid=(nblk,),
        in_specs=[
            pl.BlockSpec((2 * L, BLK), lambda i: (0, i)),
            pl.BlockSpec((2 * L, 32), lambda i: (0, 0)),
            pl.BlockSpec((32, 1), lambda i: (0, 0)),
            pl.BlockSpec((32, 3 * NG), lambda i: (0, 0)),
            pl.BlockSpec((3 * NG, 1), lambda i: (0, 0)),
        ],
        out_specs=[
            pl.BlockSpec((NG, BLK), lambda i: (0, i)),
            pl.BlockSpec((NG, BLK), lambda i: (0, i)),
            pl.BlockSpec((NG, BLK), lambda i: (0, i)),
            pl.BlockSpec((2 * L, BLK), lambda i: (0, i)),
        ],
        out_shape=[
            jax.ShapeDtypeStruct((NG, n), jnp.float32),
            jax.ShapeDtypeStruct((NG, n), jnp.float32),
            jax.ShapeDtypeStruct((NG, n), jnp.float32),
            jax.ShapeDtypeStruct((2 * L, n), jnp.float32),
        ],
    )(HT, W1, b1.reshape(32, 1), W2, b2.reshape(3 * NG, 1))


def kernel(x, hashidxs, tables, W1, b1, W2, b2):
    n = x.shape[0]
    vidx = hashidxs.astype(jnp.int32)
    x0p = jnp.zeros((NP,), jnp.float32).at[:n].set(x[:, 0])
    x1p = jnp.zeros((NP,), jnp.float32).at[:n].set(x[:, 1])
    vp = jnp.zeros((NP,), jnp.int32).at[:n].set(vidx)
    tu = lax.bitcast_convert_type(tables.astype(jnp.bfloat16), jnp.uint16)
    tpk = tu.astype(jnp.uint32)
    tab1d = lax.bitcast_convert_type(
        tpk[..., 0] | (tpk[..., 1] << 16), jnp.int32).reshape(-1)
    HT = _encode_sc(x0p, x1p, vp, tab1d)
    muT, invT, wT, hT = _mlp_tc(HT, W1, b1, W2, b2, n)
    return (muT.T, invT.T, wT.T, hT.T)


# speedup vs baseline: 1.0831x; 1.0259x over previous
import functools
import math

import jax
import jax.numpy as jnp
from jax import lax
from jax.experimental import pallas as pl
from jax.experimental.pallas import tpu as pltpu
from jax.experimental.pallas import tpu_sc as plsc

L = 8
T = 16384
FP = 2
NUM_VIEW = 16
NG = 8
HASH_K = -1640531535

NC, NS, LANES = 2, 16, 16
NW = NC * NS
C = 512
NP = 1015808
PW = NP // NW
ITERS = PW // C
IT0 = 82
IT1 = 2 * ITERS - IT0
GROUPS = C // LANES
ROWS = 4 * L * C
NSEG = 32
SEG = ROWS // NSEG

_BFAC = math.exp((math.log(512.0) - math.log(16.0)) / (L - 1))
RES = [float(math.floor(16.0 * (_BFAC ** l))) for l in range(L)]

CORNERS = ((0, 0), (0, 1), (1, 0), (1, 1))


def _encode_sc(x0p, x1p, vp, tab1d):
    mesh = plsc.VectorSubcoreMesh(core_axis_name="c", subcore_axis_name="s")

    @functools.partial(
        pl.kernel,
        mesh=mesh,
        compiler_params=pltpu.CompilerParams(needs_layout_passes=False),
        out_type=jax.ShapeDtypeStruct((2 * L, NP), jnp.float32),
        scratch_types=[
            pltpu.VMEM((2, C), jnp.float32),
            pltpu.VMEM((2, C), jnp.float32),
            pltpu.VMEM((2, C), jnp.int32),
            pltpu.VMEM((ROWS,), jnp.int32),
            pltpu.VMEM((ROWS,), jnp.int32),
            pltpu.VMEM((2, ROWS), jnp.float32),
            pltpu.VMEM((ROWS,), jnp.int32),
            pltpu.VMEM((ROWS,), jnp.int32),
            pltpu.VMEM((2, 2 * L, C), jnp.float32),
            pltpu.SemaphoreType.DMA,
            pltpu.SemaphoreType.DMA,
            pltpu.SemaphoreType.DMA,
            pltpu.SemaphoreType.DMA,
        ],
    )
    def enc(x0h, x1h, vh, tabh, outh, x0v, x1v, vv, idxv0, idxv1, wv,
            rowsv0, rowsv1, hv, gsem0, gsem1, hsem0, hsem1):
        cid = lax.axis_index("c")
        sid = lax.axis_index("s")
        my_iters = jnp.where(cid == 0, IT0, IT1)
        base0 = jnp.where(cid == 0, sid * (IT0 * C),
                          NS * (IT0 * C) + sid * (IT1 * C))
        gsems = (gsem0, gsem1)
        hsems = (hsem0, hsem1)
        idxvs = (idxv0, idxv1)
        rowsvs = (rowsv0, rowsv1)

        def phase1(k, par):
            base = base0 + k * C
            pltpu.sync_copy(x0h.at[pl.ds(base, C)], x0v.at[par])
            pltpu.sync_copy(x1h.at[pl.ds(base, C)], x1v.at[par])
            pltpu.sync_copy(vh.at[pl.ds(base, C)], vv.at[par])

            @plsc.parallel_loop(0, GROUPS, unroll=4)
            def idx_body(g):
                p0 = g * LANES
                xa = x0v[par, pl.ds(p0, LANES)]
                xb = x1v[par, pl.ds(p0, LANES)]
                vrow = vv[par, pl.ds(p0, LANES)] * (L * T)
                for l in range(L):
                    pa = xa * RES[l]
                    pb = xb * RES[l]
                    ia = pa.astype(jnp.int32)
                    ib = pb.astype(jnp.int32)
                    fa = pa - ia.astype(jnp.float32)
                    fb = pb - ib.astype(jnp.float32)
                    ga = 1.0 - fa
                    gb = 1.0 - fb
                    lb = vrow + l * T
                    for ci, (dx, dy) in enumerate(CORNERS):
                        cx = ia + dx if dx else ia
                        cy = ib + dy if dy else ib
                        h = (cx ^ (cy * HASH_K)) & (T - 1)
                        P = (l * 4 + ci) * C + p0
                        idxvs[par][pl.ds(P, LANES)] = lb + h
                        wx = fa if dx else ga
                        wy = fb if dy else gb
                        wv[par, pl.ds(P, LANES)] = wx * wy

            for j in range(NSEG):
                pltpu.async_copy(
                    tabh.at[idxvs[par].at[pl.ds(j * SEG, SEG)]],
                    rowsvs[par].at[pl.ds(j * SEG, SEG)],
                    gsems[par])

        def phase2(k, par):
            base = base0 + k * C
            for j in range(NSEG):
                pltpu.make_async_copy(
                    tabh.at[idxvs[par].at[pl.ds(j * SEG, SEG)]],
                    rowsvs[par].at[pl.ds(j * SEG, SEG)],
                    gsems[par]).wait()

            himask = jnp.full((LANES,), -65536, jnp.int32)

            def run_acc():
                @plsc.parallel_loop(0, GROUPS, unroll=4)
                def acc_body(g):
                    p0 = g * LANES
                    for l in range(L):
                        a0 = jnp.zeros((LANES,), jnp.float32)
                        a1 = jnp.zeros((LANES,), jnp.float32)
                        for ci in range(4):
                            P = (l * 4 + ci) * C + p0
                            v = rowsvs[par][pl.ds(P, LANES)]
                            f0 = plsc.bitcast(v << 16, jnp.float32)
                            f1 = plsc.bitcast(v & himask, jnp.float32)
                            w = wv[par, pl.ds(P, LANES)]
                            a0 = a0 + f0 * w
                            a1 = a1 + f1 * w
                        hv[par, 2 * l, pl.ds(p0, LANES)] = a0
                        hv[par, 2 * l + 1, pl.ds(p0, LANES)] = a1

            @pl.when(k >= 2)
            def _():
                pltpu.make_async_copy(
                    hv.at[par],
                    outh.at[:, pl.ds(base - 2 * C, C)],
                    hsems[par]).wait()

            run_acc()
            pltpu.async_copy(hv.at[par], outh.at[:, pl.ds(base, C)],
                             hsems[par])

        phase1(0, 0)

        def body(i2, carry):
            a = 2 * i2
            phase1(a + 1, 1)
            phase2(a, 0)

            @pl.when(a + 2 < my_iters)
            def _():
                phase1(a + 2, 0)

            phase2(a + 1, 1)
            return carry

        lax.fori_loop(0, my_iters // 2, body, 0)
        pltpu.make_async_copy(
            hv.at[0], outh.at[:, pl.ds(base0 + (my_iters - 2) * C, C)],
            hsems[0]).wait()
        pltpu.make_async_copy(
            hv.at[1], outh.at[:, pl.ds(base0 + (my_iters - 1) * C, C)],
            hsems[1]).wait()

    return enc(x0p, x1p, vp, tab1d)


def _mlp_tc(HT, W1, b1, W2, b2, n):
    BLK = 8192
    nblk = (n + BLK - 1) // BLK

    def mlp_body(h_ref, w1_ref, b1_ref, w2_ref, b2_ref,
                 mu_ref, inv_ref, wt_ref, ho_ref):
        hT = h_ref[...]
        h1 = lax.dot_general(w1_ref[...], hT, (((0,), (0,)), ((), ())),
                             preferred_element_type=jnp.float32)
        h1 = h1 + b1_ref[...]
        g = jnp.exp(h1 * h1 * (-50.0))
        raw = lax.dot_general(w2_ref[...], g, (((0,), (0,)), ((), ())),
                              preferred_element_type=jnp.float32)
        raw = raw + b2_ref[...]
        wt_ref[...] = jnp.exp(raw[0:NG, :])
        mu_ref[...] = jax.nn.sigmoid(raw[NG:2 * NG, :])
        inv_ref[...] = jnp.exp(raw[2 * NG:3 * NG, :])
        ho_ref[...] = hT

    return pl.pallas_call(
        mlp_body,
        grid=(nblk,),
        in_specs=[
            pl.BlockSpec((2 * L, BLK), lambda i: (0, i)),
            pl.BlockSpec((2 * L, 32), lambda i: (0, 0)),
            pl.BlockSpec((32, 1), lambda i: (0, 0)),
            pl.BlockSpec((32, 3 * NG), lambda i: (0, 0)),
            pl.BlockSpec((3 * NG, 1), lambda i: (0, 0)),
        ],
        out_specs=[
            pl.BlockSpec((NG, BLK), lambda i: (0, i)),
            pl.BlockSpec((NG, BLK), lambda i: (0, i)),
            pl.BlockSpec((NG, BLK), lambda i: (0, i)),
            pl.BlockSpec((2 * L, BLK), lambda i: (0, i)),
        ],
        out_shape=[
            jax.ShapeDtypeStruct((NG, n), jnp.float32),
            jax.ShapeDtypeStruct((NG, n), jnp.float32),
            jax.ShapeDtypeStruct((NG, n), jnp.float32),
            jax.ShapeDtypeStruct((2 * L, n), jnp.float32),
        ],
    )(HT, W1, b1.reshape(32, 1), W2, b2.reshape(3 * NG, 1))


def kernel(x, hashidxs, tables, W1, b1, W2, b2):
    n = x.shape[0]
    vidx = hashidxs.astype(jnp.int32)
    x0p = jnp.zeros((NP,), jnp.float32).at[:n].set(x[:, 0])
    x1p = jnp.zeros((NP,), jnp.float32).at[:n].set(x[:, 1])
    vp = jnp.zeros((NP,), jnp.int32).at[:n].set(vidx)
    tu = lax.bitcast_convert_type(tables.astype(jnp.bfloat16), jnp.uint16)
    tpk = tu.astype(jnp.uint32)
    tab1d = lax.bitcast_convert_type(
        tpk[..., 0] | (tpk[..., 1] << 16), jnp.int32).reshape(-1)
    HT = _encode_sc(x0p, x1p, vp, tab1d)
    muT, invT, wT, hT = _mlp_tc(HT, W1, b1, W2, b2, n)
    return (muT.T, invT.T, wT.T, hT.T)


# speedup vs baseline: 1.1110x; 1.0258x over previous
import functools
import math

import jax
import jax.numpy as jnp
from jax import lax
from jax.experimental import pallas as pl
from jax.experimental.pallas import tpu as pltpu
from jax.experimental.pallas import tpu_sc as plsc

L = 8
T = 16384
FP = 2
NUM_VIEW = 16
NG = 8
HASH_K = -1640531535

NC, NS, LANES = 2, 16, 16
NW = NC * NS
C = 512
NP = 1015808
PW = NP // NW
ITERS = PW // C
IT0 = 82
IT1 = 2 * ITERS - IT0
GROUPS = C // LANES
ROWS = 4 * L * C
NSEG = 32
SEG = ROWS // NSEG

_BFAC = math.exp((math.log(512.0) - math.log(16.0)) / (L - 1))
RES = [float(math.floor(16.0 * (_BFAC ** l))) for l in range(L)]

CORNERS = ((0, 0), (0, 1), (1, 0), (1, 1))


def _encode_sc(x0p, x1p, vp, tab1d):
    mesh = plsc.VectorSubcoreMesh(core_axis_name="c", subcore_axis_name="s")

    @functools.partial(
        pl.kernel,
        mesh=mesh,
        compiler_params=pltpu.CompilerParams(needs_layout_passes=False),
        out_type=jax.ShapeDtypeStruct((2 * L, NP), jnp.float32),
        scratch_types=[
            pltpu.VMEM((2, C), jnp.float32),
            pltpu.VMEM((2, C), jnp.float32),
            pltpu.VMEM((2, C), jnp.int32),
            pltpu.VMEM((ROWS,), jnp.int32),
            pltpu.VMEM((ROWS,), jnp.int32),
            pltpu.VMEM((2, ROWS), jnp.float32),
            pltpu.VMEM((ROWS,), jnp.int32),
            pltpu.VMEM((ROWS,), jnp.int32),
            pltpu.VMEM((2, 2 * L, C), jnp.float32),
            pltpu.SemaphoreType.DMA,
            pltpu.SemaphoreType.DMA,
            pltpu.SemaphoreType.DMA,
            pltpu.SemaphoreType.DMA,
            pltpu.SemaphoreType.DMA,
            pltpu.SemaphoreType.DMA,
        ],
    )
    def enc(x0h, x1h, vh, tabh, outh, x0v, x1v, vv, idxv0, idxv1, wv,
            rowsv0, rowsv1, hv, gsem0, gsem1, hsem0, hsem1, isem0, isem1):
        cid = lax.axis_index("c")
        sid = lax.axis_index("s")
        my_iters = jnp.where(cid == 0, IT0, IT1)
        base0 = jnp.where(cid == 0, sid * (IT0 * C),
                          NS * (IT0 * C) + sid * (IT1 * C))
        gsems = (gsem0, gsem1)
        hsems = (hsem0, hsem1)
        idxvs = (idxv0, idxv1)
        rowsvs = (rowsv0, rowsv1)
        isems = (isem0, isem1)

        def stage_async(k, par):
            base = base0 + k * C
            pltpu.async_copy(x0h.at[pl.ds(base, C)], x0v.at[par], isems[par])
            pltpu.async_copy(x1h.at[pl.ds(base, C)], x1v.at[par], isems[par])
            pltpu.async_copy(vh.at[pl.ds(base, C)], vv.at[par], isems[par])

        def stage_wait(k, par):
            base = base0 + k * C
            pltpu.make_async_copy(
                x0h.at[pl.ds(base, C)], x0v.at[par], isems[par]).wait()
            pltpu.make_async_copy(
                x1h.at[pl.ds(base, C)], x1v.at[par], isems[par]).wait()
            pltpu.make_async_copy(
                vh.at[pl.ds(base, C)], vv.at[par], isems[par]).wait()

        def phase1(k, par):
            stage_wait(k, par)

            @pl.when(k + 1 < my_iters)
            def _():
                stage_async(k + 1, 1 - par)

            @plsc.parallel_loop(0, GROUPS, unroll=4)
            def idx_body(g):
                p0 = g * LANES
                xa = x0v[par, pl.ds(p0, LANES)]
                xb = x1v[par, pl.ds(p0, LANES)]
                vrow = vv[par, pl.ds(p0, LANES)] * (L * T)
                for l in range(L):
                    pa = xa * RES[l]
                    pb = xb * RES[l]
                    ia = pa.astype(jnp.int32)
                    ib = pb.astype(jnp.int32)
                    fa = pa - ia.astype(jnp.float32)
                    fb = pb - ib.astype(jnp.float32)
                    ga = 1.0 - fa
                    gb = 1.0 - fb
                    lb = vrow + l * T
                    for ci, (dx, dy) in enumerate(CORNERS):
                        cx = ia + dx if dx else ia
                        cy = ib + dy if dy else ib
                        h = (cx ^ (cy * HASH_K)) & (T - 1)
                        P = (l * 4 + ci) * C + p0
                        idxvs[par][pl.ds(P, LANES)] = lb + h
                        wx = fa if dx else ga
                        wy = fb if dy else gb
                        wv[par, pl.ds(P, LANES)] = wx * wy

            for j in range(NSEG):
                pltpu.async_copy(
                    tabh.at[idxvs[par].at[pl.ds(j * SEG, SEG)]],
                    rowsvs[par].at[pl.ds(j * SEG, SEG)],
                    gsems[par])

        def phase2(k, par):
            base = base0 + k * C
            for j in range(NSEG):
                pltpu.make_async_copy(
                    tabh.at[idxvs[par].at[pl.ds(j * SEG, SEG)]],
                    rowsvs[par].at[pl.ds(j * SEG, SEG)],
                    gsems[par]).wait()

            himask = jnp.full((LANES,), -65536, jnp.int32)

            def run_acc():
                @plsc.parallel_loop(0, GROUPS, unroll=4)
                def acc_body(g):
                    p0 = g * LANES
                    for l in range(L):
                        a0 = jnp.zeros((LANES,), jnp.float32)
                        a1 = jnp.zeros((LANES,), jnp.float32)
                        for ci in range(4):
                            P = (l * 4 + ci) * C + p0
                            v = rowsvs[par][pl.ds(P, LANES)]
                            f0 = plsc.bitcast(v << 16, jnp.float32)
                            f1 = plsc.bitcast(v & himask, jnp.float32)
                            w = wv[par, pl.ds(P, LANES)]
                            a0 = a0 + f0 * w
                            a1 = a1 + f1 * w
                        hv[par, 2 * l, pl.ds(p0, LANES)] = a0
                        hv[par, 2 * l + 1, pl.ds(p0, LANES)] = a1

            @pl.when(k >= 2)
            def _():
                pltpu.make_async_copy(
                    hv.at[par],
                    outh.at[:, pl.ds(base - 2 * C, C)],
                    hsems[par]).wait()

            run_acc()
            pltpu.async_copy(hv.at[par], outh.at[:, pl.ds(base, C)],
                             hsems[par])

        stage_async(0, 0)
        phase1(0, 0)

        def body(i2, carry):
            a = 2 * i2
            phase1(a + 1, 1)
            phase2(a, 0)

            @pl.when(a + 2 < my_iters)
            def _():
                phase1(a + 2, 0)

            phase2(a + 1, 1)
            return carry

        lax.fori_loop(0, my_iters // 2, body, 0)
        pltpu.make_async_copy(
            hv.at[0], outh.at[:, pl.ds(base0 + (my_iters - 2) * C, C)],
            hsems[0]).wait()
        pltpu.make_async_copy(
            hv.at[1], outh.at[:, pl.ds(base0 + (my_iters - 1) * C, C)],
            hsems[1]).wait()

    return enc(x0p, x1p, vp, tab1d)


def _mlp_tc(HT, W1, b1, W2, b2, n):
    BLK = 8192
    nblk = (n + BLK - 1) // BLK

    def mlp_body(h_ref, w1_ref, b1_ref, w2_ref, b2_ref,
                 mu_ref, inv_ref, wt_ref, ho_ref):
        hT = h_ref[...]
        h1 = lax.dot_general(w1_ref[...], hT, (((0,), (0,)), ((), ())),
                             preferred_element_type=jnp.float32)
        h1 = h1 + b1_ref[...]
        g = jnp.exp(h1 * h1 * (-50.0))
        raw = lax.dot_general(w2_ref[...], g, (((0,), (0,)), ((), ())),
                              preferred_element_type=jnp.float32)
        raw = raw + b2_ref[...]
        wt_ref[...] = jnp.exp(raw[0:NG, :])
        mu_ref[...] = jax.nn.sigmoid(raw[NG:2 * NG, :])
        inv_ref[...] = jnp.exp(raw[2 * NG:3 * NG, :])
        ho_ref[...] = hT

    return pl.pallas_call(
        mlp_body,
        grid=(nblk,),
        in_specs=[
            pl.BlockSpec((2 * L, BLK), lambda i: (0, i)),
            pl.BlockSpec((2 * L, 32), lambda i: (0, 0)),
            pl.BlockSpec((32, 1), lambda i: (0, 0)),
            pl.BlockSpec((32, 3 * NG), lambda i: (0, 0)),
            pl.BlockSpec((3 * NG, 1), lambda i: (0, 0)),
        ],
        out_specs=[
            pl.BlockSpec((NG, BLK), lambda i: (0, i)),
            pl.BlockSpec((NG, BLK), lambda i: (0, i)),
            pl.BlockSpec((NG, BLK), lambda i: (0, i)),
            pl.BlockSpec((2 * L, BLK), lambda i: (0, i)),
        ],
        out_shape=[
            jax.ShapeDtypeStruct((NG, n), jnp.float32),
            jax.ShapeDtypeStruct((NG, n), jnp.float32),
            jax.ShapeDtypeStruct((NG, n), jnp.float32),
            jax.ShapeDtypeStruct((2 * L, n), jnp.float32),
        ],
    )(HT, W1, b1.reshape(32, 1), W2, b2.reshape(3 * NG, 1))


def kernel(x, hashidxs, tables, W1, b1, W2, b2):
    n = x.shape[0]
    vidx = hashidxs.astype(jnp.int32)
    x0p = jnp.zeros((NP,), jnp.float32).at[:n].set(x[:, 0])
    x1p = jnp.zeros((NP,), jnp.float32).at[:n].set(x[:, 1])
    vp = jnp.zeros((NP,), jnp.int32).at[:n].set(vidx)
    tu = lax.bitcast_convert_type(tables.astype(jnp.bfloat16), jnp.uint16)
    tpk = tu.astype(jnp.uint32)
    tab1d = lax.bitcast_convert_type(
        tpk[..., 0] | (tpk[..., 1] << 16), jnp.int32).reshape(-1)
    HT = _encode_sc(x0p, x1p, vp, tab1d)
    muT, invT, wT, hT = _mlp_tc(HT, W1, b1, W2, b2, n)
    return (muT.T, invT.T, wT.T, hT.T)


# speedup vs baseline: 1.4221x; 1.2800x over previous
import functools
import math

import jax
import jax.numpy as jnp
from jax import lax
from jax.experimental import pallas as pl
from jax.experimental.pallas import tpu as pltpu
from jax.experimental.pallas import tpu_sc as plsc

L = 8
T = 16384
FP = 2
NUM_VIEW = 16
NG = 8
HASH_K = -1640531535

NC, NS, LANES = 2, 16, 16
NW = NC * NS
C = 512
NP = 1015808
PW = NP // NW
ITERS = PW // C
IT0 = 82
IT1 = 2 * ITERS - IT0
GROUPS = C // LANES
SL = 2
NPLANES = 4 * (L - SL)
ROWS = NPLANES * C
NSEG = 24
SEG = ROWS // NSEG

_BFAC = math.exp((math.log(512.0) - math.log(16.0)) / (L - 1))
RES = [float(math.floor(16.0 * (_BFAC ** l))) for l in range(L)]

CORNERS = ((0, 0), (0, 1), (1, 0), (1, 1))

DG = [int(RES[l]) + 2 for l in range(SL)]
DOFF = [sum(g * g for g in DG[:l]) for l in range(SL)]
DPV = sum(g * g for g in DG)
DTOT = NUM_VIEW * DPV


def _encode_sc(x0p, x1p, vp, tab1d, dense):
    mesh = plsc.VectorSubcoreMesh(core_axis_name="c", subcore_axis_name="s")

    @functools.partial(
        pl.kernel,
        mesh=mesh,
        compiler_params=pltpu.CompilerParams(needs_layout_passes=False),
        out_type=jax.ShapeDtypeStruct((2 * L, NP), jnp.float32),
        scratch_types=[
            pltpu.VMEM((2, C), jnp.float32),
            pltpu.VMEM((2, C), jnp.float32),
            pltpu.VMEM((2, C), jnp.int32),
            pltpu.VMEM((ROWS,), jnp.int32),
            pltpu.VMEM((ROWS,), jnp.int32),
            pltpu.VMEM((2, ROWS), jnp.float32),
            pltpu.VMEM((ROWS,), jnp.int32),
            pltpu.VMEM((ROWS,), jnp.int32),
            pltpu.VMEM((2, 2 * L, C), jnp.float32),
            pltpu.VMEM((DTOT,), jnp.int32),
            pltpu.SemaphoreType.DMA,
            pltpu.SemaphoreType.DMA,
            pltpu.SemaphoreType.DMA,
            pltpu.SemaphoreType.DMA,
            pltpu.SemaphoreType.DMA,
            pltpu.SemaphoreType.DMA,
        ],
    )
    def enc(x0h, x1h, vh, tabh, denseh, outh, x0v, x1v, vv, idxv0, idxv1,
            wv, rowsv0, rowsv1, hv, densev, gsem0, gsem1, hsem0, hsem1,
            isem0, isem1):
        pltpu.sync_copy(denseh, densev)
        cid = lax.axis_index("c")
        sid = lax.axis_index("s")
        my_iters = jnp.where(cid == 0, IT0, IT1)
        base0 = jnp.where(cid == 0, sid * (IT0 * C),
                          NS * (IT0 * C) + sid * (IT1 * C))
        gsems = (gsem0, gsem1)
        hsems = (hsem0, hsem1)
        idxvs = (idxv0, idxv1)
        rowsvs = (rowsv0, rowsv1)
        isems = (isem0, isem1)

        def stage_async(k, par):
            base = base0 + k * C
            pltpu.async_copy(x0h.at[pl.ds(base, C)], x0v.at[par], isems[par])
            pltpu.async_copy(x1h.at[pl.ds(base, C)], x1v.at[par], isems[par])
            pltpu.async_copy(vh.at[pl.ds(base, C)], vv.at[par], isems[par])

        def stage_wait(k, par):
            base = base0 + k * C
            pltpu.make_async_copy(
                x0h.at[pl.ds(base, C)], x0v.at[par], isems[par]).wait()
            pltpu.make_async_copy(
                x1h.at[pl.ds(base, C)], x1v.at[par], isems[par]).wait()
            pltpu.make_async_copy(
                vh.at[pl.ds(base, C)], vv.at[par], isems[par]).wait()

        def phase1(k, par):
            stage_wait(k, par)

            @pl.when(k + 1 < my_iters)
            def _():
                stage_async(k + 1, 1 - par)

            base = base0 + k * C

            @pl.when(k >= 2)
            def _():
                pltpu.make_async_copy(
                    hv.at[par],
                    outh.at[:, pl.ds(base - 2 * C, C)],
                    hsems[par]).wait()

            himask1 = jnp.full((LANES,), -65536, jnp.int32)

            @plsc.parallel_loop(0, GROUPS, unroll=4)
            def idx_body(g):
                p0 = g * LANES
                xa = x0v[par, pl.ds(p0, LANES)]
                xb = x1v[par, pl.ds(p0, LANES)]
                vcol = vv[par, pl.ds(p0, LANES)]
                vrow = vcol * (L * T)
                vdens = vcol * DPV
                for l in range(SL):
                    pa = xa * RES[l]
                    pb = xb * RES[l]
                    ia = pa.astype(jnp.int32)
                    ib = pb.astype(jnp.int32)
                    fa = pa - ia.astype(jnp.float32)
                    fb = pb - ib.astype(jnp.float32)
                    ga = 1.0 - fa
                    gb = 1.0 - fb
                    db = vdens + (DOFF[l] + DG[l])
                    a0 = jnp.zeros((LANES,), jnp.float32)
                    a1 = jnp.zeros((LANES,), jnp.float32)
                    for ci, (dx, dy) in enumerate(CORNERS):
                        cx = ia + dx if dx else ia
                        cyg = (ib + dy if dy else ib) * DG[l]
                        v = plsc.load_gather(densev, [db - DG[l] + cyg + cx])
                        f0 = plsc.bitcast(v << 16, jnp.float32)
                        f1 = plsc.bitcast(v & himask1, jnp.float32)
                        wx = fa if dx else ga
                        wy = fb if dy else gb
                        w = wx * wy
                        a0 = a0 + f0 * w
                        a1 = a1 + f1 * w
                    hv[par, 2 * l, pl.ds(p0, LANES)] = a0
                    hv[par, 2 * l + 1, pl.ds(p0, LANES)] = a1
                for l in range(SL, L):
                    pa = xa * RES[l]
                    pb = xb * RES[l]
                    ia = pa.astype(jnp.int32)
                    ib = pb.astype(jnp.int32)
                    fa = pa - ia.astype(jnp.float32)
                    fb = pb - ib.astype(jnp.float32)
                    ga = 1.0 - fa
                    gb = 1.0 - fb
                    lb = vrow + l * T
                    for ci, (dx, dy) in enumerate(CORNERS):
                        cx = ia + dx if dx else ia
                        cy = ib + dy if dy else ib
                        h = (cx ^ (cy * HASH_K)) & (T - 1)
                        P = ((l - SL) * 4 + ci) * C + p0
                        idxvs[par][pl.ds(P, LANES)] = lb + h
                        wx = fa if dx else ga
                        wy = fb if dy else gb
                        wv[par, pl.ds(P, LANES)] = wx * wy

            for j in range(NSEG):
                pltpu.async_copy(
                    tabh.at[idxvs[par].at[pl.ds(j * SEG, SEG)]],
                    rowsvs[par].at[pl.ds(j * SEG, SEG)],
                    gsems[par])

        def phase2(k, par):
            base = base0 + k * C
            for j in range(NSEG):
                pltpu.make_async_copy(
                    tabh.at[idxvs[par].at[pl.ds(j * SEG, SEG)]],
                    rowsvs[par].at[pl.ds(j * SEG, SEG)],
                    gsems[par]).wait()

            himask = jnp.full((LANES,), -65536, jnp.int32)

            def run_acc():
                @plsc.parallel_loop(0, GROUPS, unroll=4)
                def acc_body(g):
                    p0 = g * LANES
                    for l in range(SL, L):
                        a0 = jnp.zeros((LANES,), jnp.float32)
                        a1 = jnp.zeros((LANES,), jnp.float32)
                        for ci in range(4):
                            P = ((l - SL) * 4 + ci) * C + p0
                            v = rowsvs[par][pl.ds(P, LANES)]
                            f0 = plsc.bitcast(v << 16, jnp.float32)
                            f1 = plsc.bitcast(v & himask, jnp.float32)
                            w = wv[par, pl.ds(P, LANES)]
                            a0 = a0 + f0 * w
                            a1 = a1 + f1 * w
                        hv[par, 2 * l, pl.ds(p0, LANES)] = a0
                        hv[par, 2 * l + 1, pl.ds(p0, LANES)] = a1

            run_acc()
            pltpu.async_copy(hv.at[par], outh.at[:, pl.ds(base, C)],
                             hsems[par])

        stage_async(0, 0)
        phase1(0, 0)

        def body(i2, carry):
            a = 2 * i2
            phase1(a + 1, 1)
            phase2(a, 0)

            @pl.when(a + 2 < my_iters)
            def _():
                phase1(a + 2, 0)

            phase2(a + 1, 1)
            return carry

        lax.fori_loop(0, my_iters // 2, body, 0)
        pltpu.make_async_copy(
            hv.at[0], outh.at[:, pl.ds(base0 + (my_iters - 2) * C, C)],
            hsems[0]).wait()
        pltpu.make_async_copy(
            hv.at[1], outh.at[:, pl.ds(base0 + (my_iters - 1) * C, C)],
            hsems[1]).wait()

    return enc(x0p, x1p, vp, tab1d, dense)


def _mlp_tc(HT, W1, b1, W2, b2, n):
    BLK = 8192
    nblk = (n + BLK - 1) // BLK

    def mlp_body(h_ref, w1_ref, b1_ref, w2_ref, b2_ref,
                 mu_ref, inv_ref, wt_ref, ho_ref):
        hT = h_ref[...]
        h1 = lax.dot_general(w1_ref[...], hT, (((0,), (0,)), ((), ())),
                             preferred_element_type=jnp.float32)
        h1 = h1 + b1_ref[...]
        g = jnp.exp(h1 * h1 * (-50.0))
        raw = lax.dot_general(w2_ref[...], g, (((0,), (0,)), ((), ())),
                              preferred_element_type=jnp.float32)
        raw = raw + b2_ref[...]
        wt_ref[...] = jnp.exp(raw[0:NG, :])
        mu_ref[...] = jax.nn.sigmoid(raw[NG:2 * NG, :])
        inv_ref[...] = jnp.exp(raw[2 * NG:3 * NG, :])
        ho_ref[...] = hT

    return pl.pallas_call(
        mlp_body,
        grid=(nblk,),
        in_specs=[
            pl.BlockSpec((2 * L, BLK), lambda i: (0, i)),
            pl.BlockSpec((2 * L, 32), lambda i: (0, 0)),
            pl.BlockSpec((32, 1), lambda i: (0, 0)),
            pl.BlockSpec((32, 3 * NG), lambda i: (0, 0)),
            pl.BlockSpec((3 * NG, 1), lambda i: (0, 0)),
        ],
        out_specs=[
            pl.BlockSpec((NG, BLK), lambda i: (0, i)),
            pl.BlockSpec((NG, BLK), lambda i: (0, i)),
            pl.BlockSpec((NG, BLK), lambda i: (0, i)),
            pl.BlockSpec((2 * L, BLK), lambda i: (0, i)),
        ],
        out_shape=[
            jax.ShapeDtypeStruct((NG, n), jnp.float32),
            jax.ShapeDtypeStruct((NG, n), jnp.float32),
            jax.ShapeDtypeStruct((NG, n), jnp.float32),
            jax.ShapeDtypeStruct((2 * L, n), jnp.float32),
        ],
    )(HT, W1, b1.reshape(32, 1), W2, b2.reshape(3 * NG, 1))


def kernel(x, hashidxs, tables, W1, b1, W2, b2):
    n = x.shape[0]
    vidx = hashidxs.astype(jnp.int32)
    x0p = jnp.zeros((NP,), jnp.float32).at[:n].set(x[:, 0])
    x1p = jnp.zeros((NP,), jnp.float32).at[:n].set(x[:, 1])
    vp = jnp.zeros((NP,), jnp.int32).at[:n].set(vidx)
    tu = lax.bitcast_convert_type(tables.astype(jnp.bfloat16), jnp.uint16)
    tpk = tu.astype(jnp.uint32)
    tab1d = lax.bitcast_convert_type(
        tpk[..., 0] | (tpk[..., 1] << 16), jnp.int32).reshape(-1)
    dparts = []
    vr = jnp.arange(NUM_VIEW, dtype=jnp.int32)[:, None]
    for l in range(SL):
        g = DG[l]
        cy = jnp.arange(g, dtype=jnp.int32)[:, None]
        cx = jnp.arange(g, dtype=jnp.int32)[None, :]
        h = ((cx ^ (cy * HASH_K)) & (T - 1)).reshape(-1)
        dparts.append(vr * (L * T) + l * T + h[None, :])
    didx = jnp.concatenate(dparts, axis=1).reshape(-1)
    dense = tab1d[didx]
    HT = _encode_sc(x0p, x1p, vp, tab1d, dense)
    muT, invT, wT, hT = _mlp_tc(HT, W1, b1, W2, b2, n)
    return (muT.T, invT.T, wT.T, hT.T)


# speedup vs baseline: 1.4236x; 1.0011x over previous
import functools
import math

import jax
import jax.numpy as jnp
from jax import lax
from jax.experimental import pallas as pl
from jax.experimental.pallas import tpu as pltpu
from jax.experimental.pallas import tpu_sc as plsc

L = 8
T = 16384
FP = 2
NUM_VIEW = 16
NG = 8
HASH_K = -1640531535

NC, NS, LANES = 2, 16, 16
NW = NC * NS
C = 512
NP = 1015808
PW = NP // NW
ITERS = PW // C
IT0 = 88
IT1 = 2 * ITERS - IT0
GROUPS = C // LANES
SL = 2
NPLANES = 4 * (L - SL)
ROWS = NPLANES * C
NSEG = 24
SEG = ROWS // NSEG

_BFAC = math.exp((math.log(512.0) - math.log(16.0)) / (L - 1))
RES = [float(math.floor(16.0 * (_BFAC ** l))) for l in range(L)]

CORNERS = ((0, 0), (0, 1), (1, 0), (1, 1))

DG = [int(RES[l]) + 2 for l in range(SL)]
DOFF = [sum(g * g for g in DG[:l]) for l in range(SL)]
DPV = sum(g * g for g in DG)
DTOT = NUM_VIEW * DPV


def _encode_sc(x0p, x1p, vp, tab1d, dense):
    mesh = plsc.VectorSubcoreMesh(core_axis_name="c", subcore_axis_name="s")

    @functools.partial(
        pl.kernel,
        mesh=mesh,
        compiler_params=pltpu.CompilerParams(needs_layout_passes=False),
        out_type=jax.ShapeDtypeStruct((2 * L, NP), jnp.float32),
        scratch_types=[
            pltpu.VMEM((2, C), jnp.float32),
            pltpu.VMEM((2, C), jnp.float32),
            pltpu.VMEM((2, C), jnp.int32),
            pltpu.VMEM((ROWS,), jnp.int32),
            pltpu.VMEM((ROWS,), jnp.int32),
            pltpu.VMEM((2, ROWS), jnp.float32),
            pltpu.VMEM((ROWS,), jnp.int32),
            pltpu.VMEM((ROWS,), jnp.int32),
            pltpu.VMEM((2, 2 * L, C), jnp.float32),
            pltpu.VMEM((DTOT,), jnp.int32),
            pltpu.SemaphoreType.DMA,
            pltpu.SemaphoreType.DMA,
            pltpu.SemaphoreType.DMA,
            pltpu.SemaphoreType.DMA,
            pltpu.SemaphoreType.DMA,
            pltpu.SemaphoreType.DMA,
        ],
    )
    def enc(x0h, x1h, vh, tabh, denseh, outh, x0v, x1v, vv, idxv0, idxv1,
            wv, rowsv0, rowsv1, hv, densev, gsem0, gsem1, hsem0, hsem1,
            isem0, isem1):
        pltpu.sync_copy(denseh, densev)
        cid = lax.axis_index("c")
        sid = lax.axis_index("s")
        my_iters = jnp.where(cid == 0, IT0, IT1)
        base0 = jnp.where(cid == 0, sid * (IT0 * C),
                          NS * (IT0 * C) + sid * (IT1 * C))
        gsems = (gsem0, gsem1)
        hsems = (hsem0, hsem1)
        idxvs = (idxv0, idxv1)
        rowsvs = (rowsv0, rowsv1)
        isems = (isem0, isem1)

        def stage_async(k, par):
            base = base0 + k * C
            pltpu.async_copy(x0h.at[pl.ds(base, C)], x0v.at[par], isems[par])
            pltpu.async_copy(x1h.at[pl.ds(base, C)], x1v.at[par], isems[par])
            pltpu.async_copy(vh.at[pl.ds(base, C)], vv.at[par], isems[par])

        def stage_wait(k, par):
            base = base0 + k * C
            pltpu.make_async_copy(
                x0h.at[pl.ds(base, C)], x0v.at[par], isems[par]).wait()
            pltpu.make_async_copy(
                x1h.at[pl.ds(base, C)], x1v.at[par], isems[par]).wait()
            pltpu.make_async_copy(
                vh.at[pl.ds(base, C)], vv.at[par], isems[par]).wait()

        def phase1(k, par):
            stage_wait(k, par)

            @pl.when(k + 1 < my_iters)
            def _():
                stage_async(k + 1, 1 - par)

            base = base0 + k * C

            @pl.when(k >= 2)
            def _():
                pltpu.make_async_copy(
                    hv.at[par],
                    outh.at[:, pl.ds(base - 2 * C, C)],
                    hsems[par]).wait()

            himask1 = jnp.full((LANES,), -65536, jnp.int32)

            @plsc.parallel_loop(0, GROUPS, unroll=4)
            def idx_body(g):
                p0 = g * LANES
                xa = x0v[par, pl.ds(p0, LANES)]
                xb = x1v[par, pl.ds(p0, LANES)]
                vcol = vv[par, pl.ds(p0, LANES)]
                vrow = vcol * (L * T)
                vdens = vcol * DPV
                for l in range(SL):
                    pa = xa * RES[l]
                    pb = xb * RES[l]
                    ia = pa.astype(jnp.int32)
                    ib = pb.astype(jnp.int32)
                    fa = pa - ia.astype(jnp.float32)
                    fb = pb - ib.astype(jnp.float32)
                    ga = 1.0 - fa
                    gb = 1.0 - fb
                    db = vdens + (DOFF[l] + DG[l])
                    a0 = jnp.zeros((LANES,), jnp.float32)
                    a1 = jnp.zeros((LANES,), jnp.float32)
                    for ci, (dx, dy) in enumerate(CORNERS):
                        cx = ia + dx if dx else ia
                        cyg = (ib + dy if dy else ib) * DG[l]
                        v = plsc.load_gather(densev, [db - DG[l] + cyg + cx])
                        f0 = plsc.bitcast(v << 16, jnp.float32)
                        f1 = plsc.bitcast(v & himask1, jnp.float32)
                        wx = fa if dx else ga
                        wy = fb if dy else gb
                        w = wx * wy
                        a0 = a0 + f0 * w
                        a1 = a1 + f1 * w
                    hv[par, 2 * l, pl.ds(p0, LANES)] = a0
                    hv[par, 2 * l + 1, pl.ds(p0, LANES)] = a1
                for l in range(SL, L):
                    pa = xa * RES[l]
                    pb = xb * RES[l]
                    ia = pa.astype(jnp.int32)
                    ib = pb.astype(jnp.int32)
                    fa = pa - ia.astype(jnp.float32)
                    fb = pb - ib.astype(jnp.float32)
                    ga = 1.0 - fa
                    gb = 1.0 - fb
                    lb = vrow + l * T
                    for ci, (dx, dy) in enumerate(CORNERS):
                        cx = ia + dx if dx else ia
                        cy = ib + dy if dy else ib
                        h = (cx ^ (cy * HASH_K)) & (T - 1)
                        P = ((l - SL) * 4 + ci) * C + p0
                        idxvs[par][pl.ds(P, LANES)] = lb + h
                        wx = fa if dx else ga
                        wy = fb if dy else gb
                        wv[par, pl.ds(P, LANES)] = wx * wy

            for j in range(NSEG):
                pltpu.async_copy(
                    tabh.at[idxvs[par].at[pl.ds(j * SEG, SEG)]],
                    rowsvs[par].at[pl.ds(j * SEG, SEG)],
                    gsems[par])

        def phase2(k, par):
            base = base0 + k * C
            for j in range(NSEG):
                pltpu.make_async_copy(
                    tabh.at[idxvs[par].at[pl.ds(j * SEG, SEG)]],
                    rowsvs[par].at[pl.ds(j * SEG, SEG)],
                    gsems[par]).wait()

            himask = jnp.full((LANES,), -65536, jnp.int32)

            def run_acc():
                @plsc.parallel_loop(0, GROUPS, unroll=4)
                def acc_body(g):
                    p0 = g * LANES
                    for l in range(SL, L):
                        a0 = jnp.zeros((LANES,), jnp.float32)
                        a1 = jnp.zeros((LANES,), jnp.float32)
                        for ci in range(4):
                            P = ((l - SL) * 4 + ci) * C + p0
                            v = rowsvs[par][pl.ds(P, LANES)]
                            f0 = plsc.bitcast(v << 16, jnp.float32)
                            f1 = plsc.bitcast(v & himask, jnp.float32)
                            w = wv[par, pl.ds(P, LANES)]
                            a0 = a0 + f0 * w
                            a1 = a1 + f1 * w
                        hv[par, 2 * l, pl.ds(p0, LANES)] = a0
                        hv[par, 2 * l + 1, pl.ds(p0, LANES)] = a1

            run_acc()
            pltpu.async_copy(hv.at[par], outh.at[:, pl.ds(base, C)],
                             hsems[par])

        stage_async(0, 0)
        phase1(0, 0)

        def body(i2, carry):
            a = 2 * i2
            phase1(a + 1, 1)
            phase2(a, 0)

            @pl.when(a + 2 < my_iters)
            def _():
                phase1(a + 2, 0)

            phase2(a + 1, 1)
            return carry

        lax.fori_loop(0, my_iters // 2, body, 0)
        pltpu.make_async_copy(
            hv.at[0], outh.at[:, pl.ds(base0 + (my_iters - 2) * C, C)],
            hsems[0]).wait()
        pltpu.make_async_copy(
            hv.at[1], outh.at[:, pl.ds(base0 + (my_iters - 1) * C, C)],
            hsems[1]).wait()

    return enc(x0p, x1p, vp, tab1d, dense)


def _mlp_tc(HT, W1, b1, W2, b2, n):
    BLK = 8192
    nblk = (n + BLK - 1) // BLK

    def mlp_body(h_ref, w1_ref, b1_ref, w2_ref, b2_ref,
                 mu_ref, inv_ref, wt_ref, ho_ref):
        hT = h_ref[...]
        h1 = lax.dot_general(w1_ref[...], hT, (((0,), (0,)), ((), ())),
                             preferred_element_type=jnp.float32)
        h1 = h1 + b1_ref[...]
        g = jnp.exp(h1 * h1 * (-50.0))
        raw = lax.dot_general(w2_ref[...], g, (((0,), (0,)), ((), ())),
                              preferred_element_type=jnp.float32)
        raw = raw + b2_ref[...]
        wt_ref[...] = jnp.exp(raw[0:NG, :])
        mu_ref[...] = jax.nn.sigmoid(raw[NG:2 * NG, :])
        inv_ref[...] = jnp.exp(raw[2 * NG:3 * NG, :])
        ho_ref[...] = hT

    return pl.pallas_call(
        mlp_body,
        grid=(nblk,),
        in_specs=[
            pl.BlockSpec((2 * L, BLK), lambda i: (0, i)),
            pl.BlockSpec((2 * L, 32), lambda i: (0, 0)),
            pl.BlockSpec((32, 1), lambda i: (0, 0)),
            pl.BlockSpec((32, 3 * NG), lambda i: (0, 0)),
            pl.BlockSpec((3 * NG, 1), lambda i: (0, 0)),
        ],
        out_specs=[
            pl.BlockSpec((NG, BLK), lambda i: (0, i)),
            pl.BlockSpec((NG, BLK), lambda i: (0, i)),
            pl.BlockSpec((NG, BLK), lambda i: (0, i)),
            pl.BlockSpec((2 * L, BLK), lambda i: (0, i)),
        ],
        out_shape=[
            jax.ShapeDtypeStruct((NG, n), jnp.float32),
            jax.ShapeDtypeStruct((NG, n), jnp.float32),
            jax.ShapeDtypeStruct((NG, n), jnp.float32),
            jax.ShapeDtypeStruct((2 * L, n), jnp.float32),
        ],
    )(HT, W1, b1.reshape(32, 1), W2, b2.reshape(3 * NG, 1))


def kernel(x, hashidxs, tables, W1, b1, W2, b2):
    n = x.shape[0]
    vidx = hashidxs.astype(jnp.int32)
    x0p = jnp.zeros((NP,), jnp.float32).at[:n].set(x[:, 0])
    x1p = jnp.zeros((NP,), jnp.float32).at[:n].set(x[:, 1])
    vp = jnp.zeros((NP,), jnp.int32).at[:n].set(vidx)
    tu = lax.bitcast_convert_type(tables.astype(jnp.bfloat16), jnp.uint16)
    tpk = tu.astype(jnp.uint32)
    tab1d = lax.bitcast_convert_type(
        tpk[..., 0] | (tpk[..., 1] << 16), jnp.int32).reshape(-1)
    dparts = []
    vr = jnp.arange(NUM_VIEW, dtype=jnp.int32)[:, None]
    for l in range(SL):
        g = DG[l]
        cy = jnp.arange(g, dtype=jnp.int32)[:, None]
        cx = jnp.arange(g, dtype=jnp.int32)[None, :]
        h = ((cx ^ (cy * HASH_K)) & (T - 1)).reshape(-1)
        dparts.append(vr * (L * T) + l * T + h[None, :])
    didx = jnp.concatenate(dparts, axis=1).reshape(-1)
    dense = tab1d[didx]
    HT = _encode_sc(x0p, x1p, vp, tab1d, dense)
    muT, invT, wT, hT = _mlp_tc(HT, W1, b1, W2, b2, n)
    return (muT.T, invT.T, wT.T, hT.T)


# speedup vs baseline: 1.9159x; 1.3458x over previous
import functools
import math

import jax
import jax.numpy as jnp
from jax import lax
from jax.experimental import pallas as pl
from jax.experimental.pallas import tpu as pltpu
from jax.experimental.pallas import tpu_sc as plsc

L = 8
T = 16384
FP = 2
NUM_VIEW = 16
NG = 8
HASH_K = -1640531535

NC, NS, LANES = 2, 16, 16
NW = NC * NS
C = 384
NP = 1007616
PW = NP // NW
ITERS = PW // C
IT0 = 108
IT1 = 2 * ITERS - IT0
GROUPS = C // LANES
SL = 3
NPLANES = 4 * (L - SL)
ROWS = NPLANES * C
NSEG = 20
SEG = ROWS // NSEG

_BFAC = math.exp((math.log(512.0) - math.log(16.0)) / (L - 1))
RES = [float(math.floor(16.0 * (_BFAC ** l))) for l in range(L)]

CORNERS = ((0, 0), (0, 1), (1, 0), (1, 1))

DG = [int(RES[l]) + 2 for l in range(SL)]
DOFF = [sum(g * g for g in DG[:l]) for l in range(SL)]
DPV = sum(g * g for g in DG)
DTOT = NUM_VIEW * DPV


def _encode_sc(x0p, x1p, vp, tab1d, dense):
    mesh = plsc.VectorSubcoreMesh(core_axis_name="c", subcore_axis_name="s")

    @functools.partial(
        pl.kernel,
        mesh=mesh,
        compiler_params=pltpu.CompilerParams(needs_layout_passes=False),
        out_type=jax.ShapeDtypeStruct((2 * L, NP), jnp.float32),
        scratch_types=[
            pltpu.VMEM((2, C), jnp.float32),
            pltpu.VMEM((2, C), jnp.float32),
            pltpu.VMEM((2, C), jnp.int32),
            pltpu.VMEM((ROWS,), jnp.int32),
            pltpu.VMEM((ROWS,), jnp.int32),
            pltpu.VMEM((2, ROWS), jnp.float32),
            pltpu.VMEM((ROWS,), jnp.int32),
            pltpu.VMEM((ROWS,), jnp.int32),
            pltpu.VMEM((2, 2 * L, C), jnp.float32),
            pltpu.VMEM((DTOT,), jnp.int32),
            pltpu.SemaphoreType.DMA,
            pltpu.SemaphoreType.DMA,
            pltpu.SemaphoreType.DMA,
            pltpu.SemaphoreType.DMA,
            pltpu.SemaphoreType.DMA,
            pltpu.SemaphoreType.DMA,
        ],
    )
    def enc(x0h, x1h, vh, tabh, denseh, outh, x0v, x1v, vv, idxv0, idxv1,
            wv, rowsv0, rowsv1, hv, densev, gsem0, gsem1, hsem0, hsem1,
            isem0, isem1):
        pltpu.sync_copy(denseh, densev)
        cid = lax.axis_index("c")
        sid = lax.axis_index("s")
        my_iters = jnp.where(cid == 0, IT0, IT1)
        base0 = jnp.where(cid == 0, sid * (IT0 * C),
                          NS * (IT0 * C) + sid * (IT1 * C))
        gsems = (gsem0, gsem1)
        hsems = (hsem0, hsem1)
        idxvs = (idxv0, idxv1)
        rowsvs = (rowsv0, rowsv1)
        isems = (isem0, isem1)

        def stage_async(k, par):
            base = base0 + k * C
            pltpu.async_copy(x0h.at[pl.ds(base, C)], x0v.at[par], isems[par])
            pltpu.async_copy(x1h.at[pl.ds(base, C)], x1v.at[par], isems[par])
            pltpu.async_copy(vh.at[pl.ds(base, C)], vv.at[par], isems[par])

        def stage_wait(k, par):
            base = base0 + k * C
            pltpu.make_async_copy(
                x0h.at[pl.ds(base, C)], x0v.at[par], isems[par]).wait()
            pltpu.make_async_copy(
                x1h.at[pl.ds(base, C)], x1v.at[par], isems[par]).wait()
            pltpu.make_async_copy(
                vh.at[pl.ds(base, C)], vv.at[par], isems[par]).wait()

        def phase1(k, par):
            stage_wait(k, par)

            @pl.when(k + 1 < my_iters)
            def _():
                stage_async(k + 1, 1 - par)

            base = base0 + k * C

            @pl.when(k >= 2)
            def _():
                pltpu.make_async_copy(
                    hv.at[par],
                    outh.at[:, pl.ds(base - 2 * C, C)],
                    hsems[par]).wait()

            himask1 = jnp.full((LANES,), -65536, jnp.int32)

            @plsc.parallel_loop(0, GROUPS, unroll=4)
            def idx_body(g):
                p0 = g * LANES
                xa = x0v[par, pl.ds(p0, LANES)]
                xb = x1v[par, pl.ds(p0, LANES)]
                vcol = vv[par, pl.ds(p0, LANES)]
                vrow = vcol * (L * T)
                vdens = vcol * DPV
                for l in range(SL):
                    pa = xa * RES[l]
                    pb = xb * RES[l]
                    ia = pa.astype(jnp.int32)
                    ib = pb.astype(jnp.int32)
                    fa = pa - ia.astype(jnp.float32)
                    fb = pb - ib.astype(jnp.float32)
                    ga = 1.0 - fa
                    gb = 1.0 - fb
                    db = vdens + (DOFF[l] + DG[l])
                    a0 = jnp.zeros((LANES,), jnp.float32)
                    a1 = jnp.zeros((LANES,), jnp.float32)
                    for ci, (dx, dy) in enumerate(CORNERS):
                        cx = ia + dx if dx else ia
                        cyg = (ib + dy if dy else ib) * DG[l]
                        v = plsc.load_gather(densev, [db - DG[l] + cyg + cx])
                        f0 = plsc.bitcast(v << 16, jnp.float32)
                        f1 = plsc.bitcast(v & himask1, jnp.float32)
                        wx = fa if dx else ga
                        wy = fb if dy else gb
                        w = wx * wy
                        a0 = a0 + f0 * w
                        a1 = a1 + f1 * w
                    hv[par, 2 * l, pl.ds(p0, LANES)] = a0
                    hv[par, 2 * l + 1, pl.ds(p0, LANES)] = a1
                for l in range(SL, L):
                    pa = xa * RES[l]
                    pb = xb * RES[l]
                    ia = pa.astype(jnp.int32)
                    ib = pb.astype(jnp.int32)
                    fa = pa - ia.astype(jnp.float32)
                    fb = pb - ib.astype(jnp.float32)
                    ga = 1.0 - fa
                    gb = 1.0 - fb
                    lb = vrow + l * T
                    for ci, (dx, dy) in enumerate(CORNERS):
                        cx = ia + dx if dx else ia
                        cy = ib + dy if dy else ib
                        h = (cx ^ (cy * HASH_K)) & (T - 1)
                        P = ((l - SL) * 4 + ci) * C + p0
                        idxvs[par][pl.ds(P, LANES)] = lb + h
                        wx = fa if dx else ga
                        wy = fb if dy else gb
                        wv[par, pl.ds(P, LANES)] = wx * wy

            for j in range(NSEG):
                pltpu.async_copy(
                    tabh.at[idxvs[par].at[pl.ds(j * SEG, SEG)]],
                    rowsvs[par].at[pl.ds(j * SEG, SEG)],
                    gsems[par])

        def phase2(k, par):
            base = base0 + k * C
            for j in range(NSEG):
                pltpu.make_async_copy(
                    tabh.at[idxvs[par].at[pl.ds(j * SEG, SEG)]],
                    rowsvs[par].at[pl.ds(j * SEG, SEG)],
                    gsems[par]).wait()

            himask = jnp.full((LANES,), -65536, jnp.int32)

            def run_acc():
                @plsc.parallel_loop(0, GROUPS, unroll=4)
                def acc_body(g):
                    p0 = g * LANES
                    for l in range(SL, L):
                        a0 = jnp.zeros((LANES,), jnp.float32)
                        a1 = jnp.zeros((LANES,), jnp.float32)
                        for ci in range(4):
                            P = ((l - SL) * 4 + ci) * C + p0
                            v = rowsvs[par][pl.ds(P, LANES)]
                            f0 = plsc.bitcast(v << 16, jnp.float32)
                            f1 = plsc.bitcast(v & himask, jnp.float32)
                            w = wv[par, pl.ds(P, LANES)]
                            a0 = a0 + f0 * w
                            a1 = a1 + f1 * w
                        hv[par, 2 * l, pl.ds(p0, LANES)] = a0
                        hv[par, 2 * l + 1, pl.ds(p0, LANES)] = a1

            run_acc()
            pltpu.async_copy(hv.at[par], outh.at[:, pl.ds(base, C)],
                             hsems[par])

        stage_async(0, 0)
        phase1(0, 0)

        def body(i2, carry):
            a = 2 * i2
            phase1(a + 1, 1)
            phase2(a, 0)

            @pl.when(a + 2 < my_iters)
            def _():
                phase1(a + 2, 0)

            phase2(a + 1, 1)
            return carry

        lax.fori_loop(0, my_iters // 2, body, 0)
        pltpu.make_async_copy(
            hv.at[0], outh.at[:, pl.ds(base0 + (my_iters - 2) * C, C)],
            hsems[0]).wait()
        pltpu.make_async_copy(
            hv.at[1], outh.at[:, pl.ds(base0 + (my_iters - 1) * C, C)],
            hsems[1]).wait()

    return enc(x0p, x1p, vp, tab1d, dense)


def _mlp_tc(HT, W1, b1, W2, b2, n):
    BLK = 8192
    nblk = (n + BLK - 1) // BLK

    def mlp_body(h_ref, w1_ref, b1_ref, w2_ref, b2_ref,
                 mu_ref, inv_ref, wt_ref, ho_ref):
        hT = h_ref[...]
        h1 = lax.dot_general(w1_ref[...], hT, (((0,), (0,)), ((), ())),
                             preferred_element_type=jnp.float32)
        h1 = h1 + b1_ref[...]
        g = jnp.exp(h1 * h1 * (-50.0))
        raw = lax.dot_general(w2_ref[...], g, (((0,), (0,)), ((), ())),
                              preferred_element_type=jnp.float32)
        raw = raw + b2_ref[...]
        wt_ref[...] = jnp.exp(raw[0:NG, :])
        mu_ref[...] = jax.nn.sigmoid(raw[NG:2 * NG, :])
        inv_ref[...] = jnp.exp(raw[2 * NG:3 * NG, :])
        ho_ref[...] = hT

    return pl.pallas_call(
        mlp_body,
        grid=(nblk,),
        in_specs=[
            pl.BlockSpec((2 * L, BLK), lambda i: (0, i)),
            pl.BlockSpec((2 * L, 32), lambda i: (0, 0)),
            pl.BlockSpec((32, 1), lambda i: (0, 0)),
            pl.BlockSpec((32, 3 * NG), lambda i: (0, 0)),
            pl.BlockSpec((3 * NG, 1), lambda i: (0, 0)),
        ],
        out_specs=[
            pl.BlockSpec((NG, BLK), lambda i: (0, i)),
            pl.BlockSpec((NG, BLK), lambda i: (0, i)),
            pl.BlockSpec((NG, BLK), lambda i: (0, i)),
            pl.BlockSpec((2 * L, BLK), lambda i: (0, i)),
        ],
        out_shape=[
            jax.ShapeDtypeStruct((NG, n), jnp.float32),
            jax.ShapeDtypeStruct((NG, n), jnp.float32),
            jax.ShapeDtypeStruct((NG, n), jnp.float32),
            jax.ShapeDtypeStruct((2 * L, n), jnp.float32),
        ],
    )(HT, W1, b1.reshape(32, 1), W2, b2.reshape(3 * NG, 1))


def kernel(x, hashidxs, tables, W1, b1, W2, b2):
    n = x.shape[0]
    vidx = hashidxs.astype(jnp.int32)
    x0p = jnp.zeros((NP,), jnp.float32).at[:n].set(x[:, 0])
    x1p = jnp.zeros((NP,), jnp.float32).at[:n].set(x[:, 1])
    vp = jnp.zeros((NP,), jnp.int32).at[:n].set(vidx)
    tu = lax.bitcast_convert_type(tables.astype(jnp.bfloat16), jnp.uint16)
    tpk = tu.astype(jnp.uint32)
    tab1d = lax.bitcast_convert_type(
        tpk[..., 0] | (tpk[..., 1] << 16), jnp.int32).reshape(-1)
    dparts = []
    vr = jnp.arange(NUM_VIEW, dtype=jnp.int32)[:, None]
    for l in range(SL):
        g = DG[l]
        cy = jnp.arange(g, dtype=jnp.int32)[:, None]
        cx = jnp.arange(g, dtype=jnp.int32)[None, :]
        h = ((cx ^ (cy * HASH_K)) & (T - 1)).reshape(-1)
        dparts.append(vr * (L * T) + l * T + h[None, :])
    didx = jnp.concatenate(dparts, axis=1).reshape(-1)
    dense = tab1d[didx]
    HT = _encode_sc(x0p, x1p, vp, tab1d, dense)
    muT, invT, wT, hT = _mlp_tc(HT, W1, b1, W2, b2, n)
    return (muT.T, invT.T, wT.T, hT.T)


# speedup vs baseline: 1.9320x; 1.0084x over previous
import functools
import math

import jax
import jax.numpy as jnp
from jax import lax
from jax.experimental import pallas as pl
from jax.experimental.pallas import tpu as pltpu
from jax.experimental.pallas import tpu_sc as plsc

L = 8
T = 16384
FP = 2
NUM_VIEW = 16
NG = 8
HASH_K = -1640531535

NC, NS, LANES = 2, 16, 16
NW = NC * NS
C = 384
NP = 1007616
PW = NP // NW
ITERS = PW // C
IT0 = 104
IT1 = 2 * ITERS - IT0
GROUPS = C // LANES
SL = 3
NPLANES = 4 * (L - SL)
ROWS = NPLANES * C
NSEG = 20
SEG = ROWS // NSEG

_BFAC = math.exp((math.log(512.0) - math.log(16.0)) / (L - 1))
RES = [float(math.floor(16.0 * (_BFAC ** l))) for l in range(L)]

CORNERS = ((0, 0), (0, 1), (1, 0), (1, 1))

DG = [int(RES[l]) + 2 for l in range(SL)]
DOFF = [sum(g * g for g in DG[:l]) for l in range(SL)]
DPV = sum(g * g for g in DG)
DTOT = NUM_VIEW * DPV


def _encode_sc(x0p, x1p, vp, tab1d, dense):
    mesh = plsc.VectorSubcoreMesh(core_axis_name="c", subcore_axis_name="s")

    @functools.partial(
        pl.kernel,
        mesh=mesh,
        compiler_params=pltpu.CompilerParams(needs_layout_passes=False),
        out_type=jax.ShapeDtypeStruct((2 * L, NP), jnp.float32),
        scratch_types=[
            pltpu.VMEM((2, C), jnp.float32),
            pltpu.VMEM((2, C), jnp.float32),
            pltpu.VMEM((2, C), jnp.int32),
            pltpu.VMEM((ROWS,), jnp.int32),
            pltpu.VMEM((ROWS,), jnp.int32),
            pltpu.VMEM((2, ROWS), jnp.float32),
            pltpu.VMEM((ROWS,), jnp.int32),
            pltpu.VMEM((ROWS,), jnp.int32),
            pltpu.VMEM((2, 2 * L, C), jnp.float32),
            pltpu.VMEM((DTOT,), jnp.int32),
            pltpu.SemaphoreType.DMA,
            pltpu.SemaphoreType.DMA,
            pltpu.SemaphoreType.DMA,
            pltpu.SemaphoreType.DMA,
            pltpu.SemaphoreType.DMA,
            pltpu.SemaphoreType.DMA,
        ],
    )
    def enc(x0h, x1h, vh, tabh, denseh, outh, x0v, x1v, vv, idxv0, idxv1,
            wv, rowsv0, rowsv1, hv, densev, gsem0, gsem1, hsem0, hsem1,
            isem0, isem1):
        pltpu.sync_copy(denseh, densev)
        cid = lax.axis_index("c")
        sid = lax.axis_index("s")
        my_iters = jnp.where(cid == 0, IT0, IT1)
        base0 = jnp.where(cid == 0, sid * (IT0 * C),
                          NS * (IT0 * C) + sid * (IT1 * C))
        gsems = (gsem0, gsem1)
        hsems = (hsem0, hsem1)
        idxvs = (idxv0, idxv1)
        rowsvs = (rowsv0, rowsv1)
        isems = (isem0, isem1)

        def stage_async(k, par):
            base = base0 + k * C
            pltpu.async_copy(x0h.at[pl.ds(base, C)], x0v.at[par], isems[par])
            pltpu.async_copy(x1h.at[pl.ds(base, C)], x1v.at[par], isems[par])
            pltpu.async_copy(vh.at[pl.ds(base, C)], vv.at[par], isems[par])

        def stage_wait(k, par):
            base = base0 + k * C
            pltpu.make_async_copy(
                x0h.at[pl.ds(base, C)], x0v.at[par], isems[par]).wait()
            pltpu.make_async_copy(
                x1h.at[pl.ds(base, C)], x1v.at[par], isems[par]).wait()
            pltpu.make_async_copy(
                vh.at[pl.ds(base, C)], vv.at[par], isems[par]).wait()

        def phase1(k, par):
            stage_wait(k, par)

            @pl.when(k + 1 < my_iters)
            def _():
                stage_async(k + 1, 1 - par)

            base = base0 + k * C

            @pl.when(k >= 2)
            def _():
                pltpu.make_async_copy(
                    hv.at[par],
                    outh.at[:, pl.ds(base - 2 * C, C)],
                    hsems[par]).wait()

            himask1 = jnp.full((LANES,), -65536, jnp.int32)

            @plsc.parallel_loop(0, GROUPS, unroll=4)
            def idx_body(g):
                p0 = g * LANES
                xa = x0v[par, pl.ds(p0, LANES)]
                xb = x1v[par, pl.ds(p0, LANES)]
                vcol = vv[par, pl.ds(p0, LANES)]
                vrow = vcol * (L * T)
                vdens = vcol * DPV
                for l in range(SL):
                    pa = xa * RES[l]
                    pb = xb * RES[l]
                    ia = pa.astype(jnp.int32)
                    ib = pb.astype(jnp.int32)
                    fa = pa - ia.astype(jnp.float32)
                    fb = pb - ib.astype(jnp.float32)
                    ga = 1.0 - fa
                    gb = 1.0 - fb
                    db = vdens + (DOFF[l] + DG[l])
                    a0 = jnp.zeros((LANES,), jnp.float32)
                    a1 = jnp.zeros((LANES,), jnp.float32)
                    for ci, (dx, dy) in enumerate(CORNERS):
                        cx = ia + dx if dx else ia
                        cyg = (ib + dy if dy else ib) * DG[l]
                        v = plsc.load_gather(densev, [db - DG[l] + cyg + cx])
                        f0 = plsc.bitcast(v << 16, jnp.float32)
                        f1 = plsc.bitcast(v & himask1, jnp.float32)
                        wx = fa if dx else ga
                        wy = fb if dy else gb
                        w = wx * wy
                        a0 = a0 + f0 * w
                        a1 = a1 + f1 * w
                    hv[par, 2 * l, pl.ds(p0, LANES)] = a0
                    hv[par, 2 * l + 1, pl.ds(p0, LANES)] = a1
                for l in range(SL, L):
                    pa = xa * RES[l]
                    pb = xb * RES[l]
                    ia = pa.astype(jnp.int32)
                    ib = pb.astype(jnp.int32)
                    fa = pa - ia.astype(jnp.float32)
                    fb = pb - ib.astype(jnp.float32)
                    ga = 1.0 - fa
                    gb = 1.0 - fb
                    lb = vrow + l * T
                    for ci, (dx, dy) in enumerate(CORNERS):
                        cx = ia + dx if dx else ia
                        cy = ib + dy if dy else ib
                        h = (cx ^ (cy * HASH_K)) & (T - 1)
                        P = ((l - SL) * 4 + ci) * C + p0
                        idxvs[par][pl.ds(P, LANES)] = lb + h
                        wx = fa if dx else ga
                        wy = fb if dy else gb
                        wv[par, pl.ds(P, LANES)] = wx * wy

            for j in range(NSEG):
                pltpu.async_copy(
                    tabh.at[idxvs[par].at[pl.ds(j * SEG, SEG)]],
                    rowsvs[par].at[pl.ds(j * SEG, SEG)],
                    gsems[par])

        def phase2(k, par):
            base = base0 + k * C
            for j in range(NSEG):
                pltpu.make_async_copy(
                    tabh.at[idxvs[par].at[pl.ds(j * SEG, SEG)]],
                    rowsvs[par].at[pl.ds(j * SEG, SEG)],
                    gsems[par]).wait()

            himask = jnp.full((LANES,), -65536, jnp.int32)

            def run_acc():
                @plsc.parallel_loop(0, GROUPS, unroll=4)
                def acc_body(g):
                    p0 = g * LANES
                    for l in range(SL, L):
                        a0 = jnp.zeros((LANES,), jnp.float32)
                        a1 = jnp.zeros((LANES,), jnp.float32)
                        for ci in range(4):
                            P = ((l - SL) * 4 + ci) * C + p0
                            v = rowsvs[par][pl.ds(P, LANES)]
                            f0 = plsc.bitcast(v << 16, jnp.float32)
                            f1 = plsc.bitcast(v & himask, jnp.float32)
                            w = wv[par, pl.ds(P, LANES)]
                            a0 = a0 + f0 * w
                            a1 = a1 + f1 * w
                        hv[par, 2 * l, pl.ds(p0, LANES)] = a0
                        hv[par, 2 * l + 1, pl.ds(p0, LANES)] = a1

            run_acc()
            pltpu.async_copy(hv.at[par], outh.at[:, pl.ds(base, C)],
                             hsems[par])

        stage_async(0, 0)
        phase1(0, 0)

        def body(i2, carry):
            a = 2 * i2
            phase1(a + 1, 1)
            phase2(a, 0)

            @pl.when(a + 2 < my_iters)
            def _():
                phase1(a + 2, 0)

            phase2(a + 1, 1)
            return carry

        lax.fori_loop(0, my_iters // 2, body, 0)
        pltpu.make_async_copy(
            hv.at[0], outh.at[:, pl.ds(base0 + (my_iters - 2) * C, C)],
            hsems[0]).wait()
        pltpu.make_async_copy(
            hv.at[1], outh.at[:, pl.ds(base0 + (my_iters - 1) * C, C)],
            hsems[1]).wait()

    return enc(x0p, x1p, vp, tab1d, dense)


def _mlp_tc(HT, W1, b1, W2, b2, n):
    BLK = 8192
    nblk = (n + BLK - 1) // BLK

    def mlp_body(h_ref, w1_ref, b1_ref, w2_ref, b2_ref,
                 mu_ref, inv_ref, wt_ref, ho_ref):
        hT = h_ref[...]
        h1 = lax.dot_general(w1_ref[...], hT, (((0,), (0,)), ((), ())),
                             preferred_element_type=jnp.float32)
        h1 = h1 + b1_ref[...]
        g = jnp.exp(h1 * h1 * (-50.0))
        raw = lax.dot_general(w2_ref[...], g, (((0,), (0,)), ((), ())),
                              preferred_element_type=jnp.float32)
        raw = raw + b2_ref[...]
        wt_ref[...] = jnp.exp(raw[0:NG, :])
        mu_ref[...] = jax.nn.sigmoid(raw[NG:2 * NG, :])
        inv_ref[...] = jnp.exp(raw[2 * NG:3 * NG, :])
        ho_ref[...] = hT

    return pl.pallas_call(
        mlp_body,
        grid=(nblk,),
        in_specs=[
            pl.BlockSpec((2 * L, BLK), lambda i: (0, i)),
            pl.BlockSpec((2 * L, 32), lambda i: (0, 0)),
            pl.BlockSpec((32, 1), lambda i: (0, 0)),
            pl.BlockSpec((32, 3 * NG), lambda i: (0, 0)),
            pl.BlockSpec((3 * NG, 1), lambda i: (0, 0)),
        ],
        out_specs=[
            pl.BlockSpec((NG, BLK), lambda i: (0, i)),
            pl.BlockSpec((NG, BLK), lambda i: (0, i)),
            pl.BlockSpec((NG, BLK), lambda i: (0, i)),
            pl.BlockSpec((2 * L, BLK), lambda i: (0, i)),
        ],
        out_shape=[
            jax.ShapeDtypeStruct((NG, n), jnp.float32),
            jax.ShapeDtypeStruct((NG, n), jnp.float32),
            jax.ShapeDtypeStruct((NG, n), jnp.float32),
            jax.ShapeDtypeStruct((2 * L, n), jnp.float32),
        ],
    )(HT, W1, b1.reshape(32, 1), W2, b2.reshape(3 * NG, 1))


def kernel(x, hashidxs, tables, W1, b1, W2, b2):
    n = x.shape[0]
    vidx = hashidxs.astype(jnp.int32)
    x0p = jnp.zeros((NP,), jnp.float32).at[:n].set(x[:, 0])
    x1p = jnp.zeros((NP,), jnp.float32).at[:n].set(x[:, 1])
    vp = jnp.zeros((NP,), jnp.int32).at[:n].set(vidx)
    tu = lax.bitcast_convert_type(tables.astype(jnp.bfloat16), jnp.uint16)
    tpk = tu.astype(jnp.uint32)
    tab1d = lax.bitcast_convert_type(
        tpk[..., 0] | (tpk[..., 1] << 16), jnp.int32).reshape(-1)
    dparts = []
    vr = jnp.arange(NUM_VIEW, dtype=jnp.int32)[:, None]
    for l in range(SL):
        g = DG[l]
        cy = jnp.arange(g, dtype=jnp.int32)[:, None]
        cx = jnp.arange(g, dtype=jnp.int32)[None, :]
        h = ((cx ^ (cy * HASH_K)) & (T - 1)).reshape(-1)
        dparts.append(vr * (L * T) + l * T + h[None, :])
    didx = jnp.concatenate(dparts, axis=1).reshape(-1)
    dense = tab1d[didx]
    HT = _encode_sc(x0p, x1p, vp, tab1d, dense)
    muT, invT, wT, hT = _mlp_tc(HT, W1, b1, W2, b2, n)
    return (muT.T, invT.T, wT.T, hT.T)
